# baseline (speedup 1.0000x reference)
import functools

import jax
import jax.numpy as jnp
from jax import lax
from jax.experimental import pallas as pl
from jax.experimental.pallas import tpu as pltpu
from jax.experimental.pallas import tpu_sc as plsc

_NC = 2
_NS = 16
_NW = _NC * _NS
_CH = 128

_INTERPRET = False


def _cdiv(a, b):
    return (a + b - 1) // b


_GDEPTH = 8


def _sc_gather(table, posd, srcp, dstp):
    epad = srcp.shape[0]
    per_w = epad // _NW
    nchunks = per_w // _CH
    ngroups = nchunks // _GDEPTH
    src2 = srcp.reshape(-1, _CH)
    dst2 = dstp.reshape(-1, _CH)
    mesh = plsc.VectorSubcoreMesh(core_axis_name="c", subcore_axis_name="s")

    def body(table_h, posd_h, src_h, dst_h, gsrc_h, gdst_h,
             idx_s, idx_d, bufs, bufd, gsem, wsem):
        wid = lax.axis_index("s") * _NC + lax.axis_index("c")
        base = wid * per_w
        crow = wid * nchunks
        pltpu.sync_copy(src_h.at[pl.ds(crow, nchunks)], idx_s)
        pltpu.sync_copy(dst_h.at[pl.ds(crow, nchunks)], idx_d)
        wdescs = []
        for g in range(ngroups):
            for d in wdescs:
                d.wait()
            wdescs = []
            gdescs = []
            for b in range(_GDEPTH):
                j = g * _GDEPTH + b
                gdescs.append(pltpu.async_copy(
                    table_h.at[idx_s.at[j]], bufs.at[b], gsem))
                gdescs.append(pltpu.async_copy(
                    posd_h.at[idx_d.at[j]], bufd.at[b], gsem))
            for d in gdescs:
                d.wait()
            for b in range(_GDEPTH):
                off = base + (g * _GDEPTH + b) * _CH
                wdescs.append(pltpu.async_copy(
                    bufs.at[b], gsrc_h.at[pl.ds(off, _CH)], wsem))
                wdescs.append(pltpu.async_copy(
                    bufd.at[b], gdst_h.at[pl.ds(off, _CH)], wsem))
        for d in wdescs:
            d.wait()

    f = pl.kernel(
        body,
        out_type=(jax.ShapeDtypeStruct((epad, 32), jnp.float32),
                  jax.ShapeDtypeStruct((epad, 16), jnp.float32)),
        mesh=mesh,
        scratch_types=[
            pltpu.VMEM((nchunks, _CH), jnp.int32),
            pltpu.VMEM((nchunks, _CH), jnp.int32),
            pltpu.VMEM((_GDEPTH, _CH, 32), jnp.float32),
            pltpu.VMEM((_GDEPTH, _CH, 16), jnp.float32),
            pltpu.SemaphoreType.DMA,
            pltpu.SemaphoreType.DMA,
        ],
        compiler_params=pltpu.CompilerParams(use_tc_tiling_on_sc=False),
        interpret=_INTERPRET,
    )
    return f(table, posd, src2, dst2)


def _sc_scatupd(payload, dstp, zinit, table, wr, brv):
    epad = payload.shape[0]
    n2 = table.shape[0]
    p_half = n2 // 2
    acc_rows = zinit.shape[0]
    rpt_z = acc_rows // _NS
    rpt_u = p_half // _NS
    nch_t = (epad // _CH) // _NS
    mesh = plsc.VectorSubcoreMesh(core_axis_name="c", subcore_axis_name="s")
    dst2 = dstp.reshape(-1, _CH)

    def body(pay_h, dst_h, z_h, tab_h, wr_h, br_h, tabo_h, poso_h,
             idx_all, idx_t, pay_v, wbuf, brbuf, ubuf, tbuf, tobuf, pobuf,
             psem, accum):
        cid = lax.axis_index("c")
        sid = lax.axis_index("s")
        base_node = cid * p_half
        pltpu.sync_copy(z_h.at[pl.ds(sid * rpt_z, rpt_z)],
                        accum.at[pl.ds(sid * rpt_z, rpt_z)])
        pltpu.sync_copy(dst_h.at[pl.ds(sid * nch_t, nch_t)], idx_all)
        pltpu.sync_copy(wr_h, wbuf)
        pltpu.sync_copy(br_h, brbuf)
        plsc.subcore_barrier()
        prev = pltpu.async_copy(pay_h.at[pl.ds(sid * nch_t * _CH, _CH)],
                                pay_v.at[0], psem)
        for i in range(nch_t):
            nxt = None
            if i + 1 < nch_t:
                nxt = pltpu.async_copy(
                    pay_h.at[pl.ds((sid * nch_t + i + 1) * _CH, _CH)],
                    pay_v.at[(i + 1) % 2], psem)
            for m in range(_CH // 16):
                v = idx_all[i, pl.ds(m * 16, 16)]
                loc = v - base_node
                ok = (loc >= 0) & (loc < p_half)
                idx_t[pl.ds(m * 16, 16)] = jnp.where(ok, loc, p_half)
            prev.wait()
            pltpu.sync_copy(pay_v.at[i % 2], accum.at[idx_t], add=True)
            prev = nxt
        plsc.subcore_barrier()
        lbase = sid * rpt_u
        pltpu.sync_copy(accum.at[pl.ds(lbase, rpt_u)], ubuf)
        pltpu.sync_copy(tab_h.at[pl.ds(base_node + lbase, rpt_u)], tbuf)
        mask3 = lax.broadcasted_iota(jnp.int32, (16,), 0) < 3

        def row(r, carry):
            h = tbuf[r, pl.ds(0, 16)]
            posv = tbuf[r, pl.ds(16, 16)]
            ah = ubuf[r, pl.ds(0, 16)]
            aa = ubuf[r, pl.ds(16, 16)]
            invv = 1.0 / jnp.maximum(aa, 1.0)
            inv = invv[3]
            acc = brbuf[...] + ah * inv
            for k in range(16):
                acc = acc + h[k] * wbuf[k, pl.ds(0, 16)]
            hn = h + acc
            posn = jnp.where(mask3, posv + aa * inv, 0.0)
            tobuf[r, pl.ds(0, 16)] = hn
            tobuf[r, pl.ds(16, 16)] = posn
            pobuf[r, pl.ds(0, 16)] = posn
            return carry

        lax.fori_loop(0, rpt_u, row, 0)
        pltpu.sync_copy(tobuf, tabo_h.at[pl.ds(base_node + lbase, rpt_u)])
        pltpu.sync_copy(pobuf, poso_h.at[pl.ds(base_node + lbase, rpt_u)])

    f = pl.kernel(
        body,
        out_type=(jax.ShapeDtypeStruct((n2, 32), jnp.float32),
                  jax.ShapeDtypeStruct((n2, 16), jnp.float32)),
        mesh=mesh,
        scratch_types=[
            pltpu.VMEM((nch_t, _CH), jnp.int32),
            pltpu.VMEM((_CH,), jnp.int32),
            pltpu.VMEM((2, _CH, 32), jnp.float32),
            pltpu.VMEM((16, 16), jnp.float32),
            pltpu.VMEM((16,), jnp.float32),
            pltpu.VMEM((rpt_u, 32), jnp.float32),
            pltpu.VMEM((rpt_u, 32), jnp.float32),
            pltpu.VMEM((rpt_u, 32), jnp.float32),
            pltpu.VMEM((rpt_u, 16), jnp.float32),
            pltpu.SemaphoreType.DMA,
            pltpu.VMEM_SHARED((acc_rows, 32), jnp.float32),
        ],
        compiler_params=pltpu.CompilerParams(use_tc_tiling_on_sc=False),
        interpret=_INTERPRET,
    )
    return f(payload, dst2, zinit, table, wr, brv)


def _sigmoid(v):
    return 1.0 / (1.0 + jnp.exp(-v))


def _tc_init(x, pos, W1, b1r, tn):
    n = x.shape[0]
    din = x.shape[1]
    grid = (n // tn,)

    def body(x_r, p_r, w_r, b_r, tab_r, posd_r):
        h = jnp.dot(x_r[...], w_r[...]) + b_r[...]
        p = p_r[...]
        z13 = jnp.zeros((tn, 13), jnp.float32)
        tab_r[...] = jnp.concatenate([h, p, z13], axis=1)
        posd_r[...] = jnp.concatenate([p, z13], axis=1)

    return pl.pallas_call(
        body,
        grid=grid,
        in_specs=[
            pl.BlockSpec((tn, din), lambda i: (i, 0)),
            pl.BlockSpec((tn, 3), lambda i: (i, 0)),
            pl.BlockSpec(W1.shape, lambda i: (0, 0)),
            pl.BlockSpec(b1r.shape, lambda i: (0, 0)),
        ],
        out_specs=[
            pl.BlockSpec((tn, 32), lambda i: (i, 0)),
            pl.BlockSpec((tn, 16), lambda i: (i, 0)),
        ],
        out_shape=[
            jax.ShapeDtypeStruct((n, 32), jnp.float32),
            jax.ShapeDtypeStruct((n, 16), jnp.float32),
        ],
        interpret=_INTERPRET,
    )(x, pos, W1, b1r)


def _tc_edgepre(eap, nW1a, nb1r, teb):
    epad = eap.shape[0]
    dedge = eap.shape[1]
    grid = (epad // teb,)

    def body(ea_r, w1a_r, b1_r, out_r):
        out_r[...] = jnp.dot(ea_r[...], w1a_r[...]) + b1_r[...]

    return pl.pallas_call(
        body,
        grid=grid,
        in_specs=[
            pl.BlockSpec((teb, dedge), lambda i: (i, 0)),
            pl.BlockSpec(nW1a.shape, lambda i: (0, 0)),
            pl.BlockSpec(nb1r.shape, lambda i: (0, 0)),
        ],
        out_specs=pl.BlockSpec((teb, 32), lambda i: (i, 0)),
        out_shape=jax.ShapeDtypeStruct((epad, 32), jnp.float32),
        interpret=_INTERPRET,
    )(eap, nW1a, nb1r)


def _tc_edge(gsrc, gdst, ea1, nW1b, nW2, nb2d, tile_m, sel_m, wc_l, teb):
    epad = gsrc.shape[0]
    grid = (epad // teb,)

    def body(gs_r, gd_r, ea_r, w1b_r, w2_r, b2d_r, tl_r, sl_r,
             wc_r, out_r):
        gs = gs_r[...]
        hs = gs[:, 0:16]
        rel = gs[:, 16:19] - gd_r[...][:, 0:3]
        dist = jnp.sqrt(jnp.sum(rel * rel, axis=1, keepdims=True) + 1e-12)
        zpre = ea_r[...] + dist * w1b_r[...]
        z = zpre * _sigmoid(zpre)
        wef = jnp.dot(z, w2_r[...])
        hst = jnp.dot(hs, tl_r[...])
        msg = jnp.dot(wef * hst, sl_r[...]) + jnp.dot(hs, b2d_r[...])
        cw = jnp.dot(msg, wc_r[...])
        wmsg = rel * cw
        ones = jnp.ones((teb, 1), jnp.float32)
        pad = jnp.zeros((teb, 12), jnp.float32)
        out_r[...] = jnp.concatenate([msg, wmsg, ones, pad], axis=1)

    return pl.pallas_call(
        body,
        grid=grid,
        in_specs=[
            pl.BlockSpec((teb, 32), lambda i: (i, 0)),
            pl.BlockSpec((teb, 16), lambda i: (i, 0)),
            pl.BlockSpec((teb, 32), lambda i: (i, 0)),
            pl.BlockSpec(nW1b.shape, lambda i: (0, 0)),
            pl.BlockSpec(nW2.shape, lambda i: (0, 0)),
            pl.BlockSpec(nb2d.shape, lambda i: (0, 0)),
            pl.BlockSpec(tile_m.shape, lambda i: (0, 0)),
            pl.BlockSpec(sel_m.shape, lambda i: (0, 0)),
            pl.BlockSpec(wc_l.shape, lambda i: (0, 0)),
        ],
        out_specs=pl.BlockSpec((teb, 32), lambda i: (i, 0)),
        out_shape=jax.ShapeDtypeStruct((epad, 32), jnp.float32),
        compiler_params=pltpu.CompilerParams(
            dimension_semantics=("parallel",)),
        interpret=_INTERPRET,
    )(gsrc, gdst, ea1, nW1b, nW2, nb2d, tile_m, sel_m, wc_l)


def _tc_set2set(table, batch2, W_ih, W_hh, b_lstm_r, Wo1, bo1r, Wo2, bo2r,
                bgraph, msteps):
    n = table.shape[0]
    hdim = 16

    def body(tab_r, bat_r, wih_r, whh_r, bl_r, wo1_r, bo1_r, wo2_r, bo2_r,
             out_r):
        h = tab_r[...][:, 0:hdim]
        bat = bat_r[...]
        ids = lax.broadcasted_iota(jnp.int32, (1, bgraph), 1)
        pm = bat == ids
        qstar = jnp.zeros((bgraph, 2 * hdim), jnp.float32)
        hs = jnp.zeros((bgraph, hdim), jnp.float32)
        cs = jnp.zeros((bgraph, hdim), jnp.float32)
        for _ in range(msteps):
            gates = (jnp.dot(qstar, wih_r[...]) + jnp.dot(hs, whh_r[...])
                     + bl_r[...])
            gi = gates[:, 0:hdim]
            gf = gates[:, hdim:2 * hdim]
            gg = gates[:, 2 * hdim:3 * hdim]
            go = gates[:, 3 * hdim:4 * hdim]
            cs = _sigmoid(gf) * cs + _sigmoid(gi) * jnp.tanh(gg)
            hs = _sigmoid(go) * jnp.tanh(cs)
            s = lax.dot_general(h, hs, (((1,), (1,)), ((), ())))
            masked = jnp.where(pm, s, -jnp.inf)
            emax = jnp.max(masked, axis=0, keepdims=True)
            emax = jnp.where(emax > -jnp.inf, emax, 0.0)
            a = jnp.exp(masked - emax)
            asum = jnp.sum(a, axis=0, keepdims=True)
            asum = jnp.where(asum > 0.0, asum, 1.0)
            an = a / asum
            r = lax.dot_general(an, h, (((0,), (0,)), ((), ())))
            qstar = jnp.concatenate([hs, r], axis=1)
        t = jnp.dot(qstar, wo1_r[...]) + bo1_r[...]
        sil = t * _sigmoid(t)
        out_r[...] = jnp.dot(sil, wo2_r[...]) + bo2_r[...]

    return pl.pallas_call(
        body,
        out_shape=jax.ShapeDtypeStruct((bgraph, 1), jnp.float32),
        interpret=_INTERPRET,
    )(table, batch2, W_ih, W_hh, b_lstm_r, Wo1, bo1r, Wo2, bo2r)


def kernel(x, edge_index, edge_attr, pos, batch, W1, b1, nW1, nb1, nW2, nb2,
           Wr, br, Wc, W_ih, W_hh, b_lstm, Wo1, bo1, Wo2, bo2):
    n, din = x.shape
    e = edge_index.shape[1]
    dedge = edge_attr.shape[1]
    hdim = W1.shape[1]
    nlayers = Wr.shape[0]
    bgraph = 64
    msteps = 3
    tn = 2048

    epad = _cdiv(e, _NW * _CH) * (_NW * _CH)
    n2 = _cdiv(_cdiv(n, 2 * _NS) * (2 * _NS), tn) * tn
    zpadn = n2 - n

    src = edge_index[0].astype(jnp.int32)
    dst = edge_index[1].astype(jnp.int32)
    srcp = jnp.concatenate([src, jnp.zeros((epad - e,), jnp.int32)])
    dstp = jnp.concatenate([dst, jnp.full((epad - e,), n, jnp.int32)])
    eap = jnp.concatenate(
        [edge_attr, jnp.zeros((epad - e, dedge), jnp.float32)], axis=0)
    zinit = jnp.zeros((n2 // 2 + 16, 32), jnp.float32)
    xp = jnp.concatenate([x, jnp.zeros((zpadn, din), jnp.float32)])
    posp = jnp.concatenate([pos, jnp.zeros((zpadn, 3), jnp.float32)])
    batchp = jnp.concatenate(
        [batch.astype(jnp.int32), jnp.full((zpadn,), bgraph, jnp.int32)])

    eye = jnp.eye(hdim, dtype=jnp.float32)
    tile_m = jnp.tile(eye, (1, hdim))
    sel_m = jnp.repeat(eye, hdim, axis=0)

    nW1a = nW1[:dedge]
    nW1b = nW1[dedge:dedge + 1]
    nb1r = nb1.reshape(1, -1)
    nb2d = nb2.reshape(hdim, hdim).T

    table, posd = _tc_init(xp, posp, W1, b1.reshape(1, -1), tn)
    ea1 = _tc_edgepre(eap, nW1a, nb1r, 2048)
    for l in range(nlayers):
        gsrc, gdst = _sc_gather(table, posd, srcp, dstp)
        payload = _tc_edge(gsrc, gdst, ea1, nW1b, nW2, nb2d,
                           tile_m, sel_m, Wc[l], 4096)
        table, posd = _sc_scatupd(payload, dstp, zinit, table, Wr[l], br[l])

    out = _tc_set2set(table, batchp.reshape(-1, 1),
                      W_ih, W_hh, b_lstm.reshape(1, -1), Wo1,
                      bo1.reshape(1, -1), Wo2, bo2.reshape(1, 1),
                      bgraph, msteps)
    return out.reshape(-1)


# speedup vs baseline: 1.1677x; 1.1677x over previous
import functools

import jax
import jax.numpy as jnp
from jax import lax
from jax.experimental import pallas as pl
from jax.experimental.pallas import tpu as pltpu
from jax.experimental.pallas import tpu_sc as plsc

_NC = 2
_NS = 16
_NW = _NC * _NS
_CH = 128

_INTERPRET = False


def _cdiv(a, b):
    return (a + b - 1) // b


_GDEPTH = 10


def _sc_gather(table, posd, srcp, dstp):
    epad = srcp.shape[0]
    per_w = epad // _NW
    nchunks = per_w // _CH
    src2 = srcp.reshape(-1, _CH)
    dst2 = dstp.reshape(-1, _CH)
    mesh = plsc.VectorSubcoreMesh(core_axis_name="c", subcore_axis_name="s")

    def body(table_h, posd_h, src_h, dst_h, gsrc_h, gdst_h,
             idx_s, idx_d, bufs, bufd, gsem, wsem):
        wid = lax.axis_index("s") * _NC + lax.axis_index("c")
        base = wid * per_w
        crow = wid * nchunks
        pltpu.sync_copy(src_h.at[pl.ds(crow, nchunks)], idx_s)
        pltpu.sync_copy(dst_h.at[pl.ds(crow, nchunks)], idx_d)
        lag = _GDEPTH // 2
        gd = [None] * nchunks
        wd = [None] * nchunks

        def fire_write(k):
            s = k % _GDEPTH
            gd[k][0].wait()
            gd[k][1].wait()
            off = base + k * _CH
            wd[k] = (
                pltpu.async_copy(bufs.at[s], gsrc_h.at[pl.ds(off, _CH)],
                                 wsem),
                pltpu.async_copy(bufd.at[s], gdst_h.at[pl.ds(off, _CH)],
                                 wsem),
            )

        for j in range(nchunks):
            s = j % _GDEPTH
            if j >= _GDEPTH:
                wd[j - _GDEPTH][0].wait()
                wd[j - _GDEPTH][1].wait()
            gd[j] = (
                pltpu.async_copy(table_h.at[idx_s.at[j]], bufs.at[s], gsem),
                pltpu.async_copy(posd_h.at[idx_d.at[j]], bufd.at[s], gsem),
            )
            if j >= lag:
                fire_write(j - lag)
        for k in range(nchunks - lag, nchunks):
            fire_write(k)
        for k in range(max(0, nchunks - _GDEPTH), nchunks):
            wd[k][0].wait()
            wd[k][1].wait()

    f = pl.kernel(
        body,
        out_type=(jax.ShapeDtypeStruct((epad, 32), jnp.float32),
                  jax.ShapeDtypeStruct((epad, 16), jnp.float32)),
        mesh=mesh,
        scratch_types=[
            pltpu.VMEM((nchunks, _CH), jnp.int32),
            pltpu.VMEM((nchunks, _CH), jnp.int32),
            pltpu.VMEM((_GDEPTH, _CH, 32), jnp.float32),
            pltpu.VMEM((_GDEPTH, _CH, 16), jnp.float32),
            pltpu.SemaphoreType.DMA,
            pltpu.SemaphoreType.DMA,
        ],
        compiler_params=pltpu.CompilerParams(use_tc_tiling_on_sc=False),
        interpret=_INTERPRET,
    )
    return f(table, posd, src2, dst2)


def _sc_scatter(payload, dstp, zinit):
    epad = payload.shape[0]
    nacc = zinit.shape[0]
    per_w = epad // _NW
    nchunks = per_w // _CH
    rpt = nacc // _NS
    mesh = plsc.VectorSubcoreMesh(core_axis_name="c", subcore_axis_name="s")

    dst2 = dstp.reshape(-1, _CH)

    def body(pay_h, dst_h, z_h, out_h, idx_v, pay_v, psem, accum):
        cid = lax.axis_index("c")
        sid = lax.axis_index("s")
        wid = sid * _NC + cid
        base = wid * per_w
        pltpu.sync_copy(z_h.at[pl.ds(sid * rpt, rpt)],
                        accum.at[pl.ds(sid * rpt, rpt)])
        pltpu.sync_copy(dst_h.at[pl.ds(wid * nchunks, nchunks)], idx_v)
        plsc.subcore_barrier()
        prev = pltpu.async_copy(pay_h.at[pl.ds(base, _CH)],
                                pay_v.at[0], psem)
        for i in range(nchunks):
            nxt = None
            if i + 1 < nchunks:
                nxt = pltpu.async_copy(
                    pay_h.at[pl.ds(base + (i + 1) * _CH, _CH)],
                    pay_v.at[(i + 1) % 2], psem)
            prev.wait()
            pltpu.sync_copy(pay_v.at[i % 2], accum.at[idx_v.at[i]], add=True)
            prev = nxt
        plsc.subcore_barrier()
        pltpu.sync_copy(accum.at[pl.ds(sid * rpt, rpt)],
                        out_h.at[cid, pl.ds(sid * rpt, rpt)])

    f = pl.kernel(
        body,
        out_type=jax.ShapeDtypeStruct((_NC, nacc, 32), jnp.float32),
        mesh=mesh,
        scratch_types=[
            pltpu.VMEM((nchunks, _CH), jnp.int32),
            pltpu.VMEM((2, _CH, 32), jnp.float32),
            pltpu.SemaphoreType.DMA,
            pltpu.VMEM_SHARED((nacc, 32), jnp.float32),
        ],
        compiler_params=pltpu.CompilerParams(use_tc_tiling_on_sc=False),
        interpret=_INTERPRET,
    )
    return f(payload, dst2, zinit)


def _sigmoid(v):
    return 1.0 / (1.0 + jnp.exp(-v))


def _tc_init(x, pos, W1, b1r, tn):
    n = x.shape[0]
    din = x.shape[1]
    grid = (n // tn,)

    def body(x_r, p_r, w_r, b_r, tab_r, posd_r):
        h = jnp.dot(x_r[...], w_r[...]) + b_r[...]
        p = p_r[...]
        z13 = jnp.zeros((tn, 13), jnp.float32)
        tab_r[...] = jnp.concatenate([h, p, z13], axis=1)
        posd_r[...] = jnp.concatenate([p, z13], axis=1)

    return pl.pallas_call(
        body,
        grid=grid,
        in_specs=[
            pl.BlockSpec((tn, din), lambda i: (i, 0)),
            pl.BlockSpec((tn, 3), lambda i: (i, 0)),
            pl.BlockSpec(W1.shape, lambda i: (0, 0)),
            pl.BlockSpec(b1r.shape, lambda i: (0, 0)),
        ],
        out_specs=[
            pl.BlockSpec((tn, 32), lambda i: (i, 0)),
            pl.BlockSpec((tn, 16), lambda i: (i, 0)),
        ],
        out_shape=[
            jax.ShapeDtypeStruct((n, 32), jnp.float32),
            jax.ShapeDtypeStruct((n, 16), jnp.float32),
        ],
        interpret=_INTERPRET,
    )(x, pos, W1, b1r)


def _tc_edgepre(eap, nW1a, nb1r, teb):
    epad = eap.shape[0]
    dedge = eap.shape[1]
    grid = (epad // teb,)

    def body(ea_r, w1a_r, b1_r, out_r):
        out_r[...] = jnp.dot(ea_r[...], w1a_r[...]) + b1_r[...]

    return pl.pallas_call(
        body,
        grid=grid,
        in_specs=[
            pl.BlockSpec((teb, dedge), lambda i: (i, 0)),
            pl.BlockSpec(nW1a.shape, lambda i: (0, 0)),
            pl.BlockSpec(nb1r.shape, lambda i: (0, 0)),
        ],
        out_specs=pl.BlockSpec((teb, 32), lambda i: (i, 0)),
        out_shape=jax.ShapeDtypeStruct((epad, 32), jnp.float32),
        interpret=_INTERPRET,
    )(eap, nW1a, nb1r)


def _tc_edge(gsrc, gdst, ea1, nW1b, nW2, nb2d, tile_m, sel_m, wc_l, teb):
    epad = gsrc.shape[0]
    grid = (epad // teb,)

    def body(gs_r, gd_r, ea_r, w1b_r, w2_r, b2d_r, tl_r, sl_r,
             wc_r, out_r):
        gs = gs_r[...]
        hs = gs[:, 0:16]
        rel = gs[:, 16:19] - gd_r[...][:, 0:3]
        dist = jnp.sqrt(jnp.sum(rel * rel, axis=1, keepdims=True) + 1e-12)
        zpre = ea_r[...] + dist * w1b_r[...]
        z = zpre * _sigmoid(zpre)
        wef = jnp.dot(z, w2_r[...])
        hst = jnp.dot(hs, tl_r[...])
        msg = jnp.dot(wef * hst, sl_r[...]) + jnp.dot(hs, b2d_r[...])
        cw = jnp.dot(msg, wc_r[...])
        wmsg = rel * cw
        ones = jnp.ones((teb, 1), jnp.float32)
        pad = jnp.zeros((teb, 12), jnp.float32)
        out_r[...] = jnp.concatenate([msg, wmsg, ones, pad], axis=1)

    return pl.pallas_call(
        body,
        grid=grid,
        in_specs=[
            pl.BlockSpec((teb, 32), lambda i: (i, 0)),
            pl.BlockSpec((teb, 16), lambda i: (i, 0)),
            pl.BlockSpec((teb, 32), lambda i: (i, 0)),
            pl.BlockSpec(nW1b.shape, lambda i: (0, 0)),
            pl.BlockSpec(nW2.shape, lambda i: (0, 0)),
            pl.BlockSpec(nb2d.shape, lambda i: (0, 0)),
            pl.BlockSpec(tile_m.shape, lambda i: (0, 0)),
            pl.BlockSpec(sel_m.shape, lambda i: (0, 0)),
            pl.BlockSpec(wc_l.shape, lambda i: (0, 0)),
        ],
        out_specs=pl.BlockSpec((teb, 32), lambda i: (i, 0)),
        out_shape=jax.ShapeDtypeStruct((epad, 32), jnp.float32),
        compiler_params=pltpu.CompilerParams(
            dimension_semantics=("parallel",)),
        interpret=_INTERPRET,
    )(gsrc, gdst, ea1, nW1b, nW2, nb2d, tile_m, sel_m, wc_l)


def _tc_update(table, agg2, Wr_l, br_l, tn):
    n = table.shape[0]
    nacc = agg2.shape[1]
    grid = (n // tn,)

    def body(tab_r, agg_r, wr_r, br_r, tabo_r, posdo_r):
        agg = agg_r[0] + agg_r[1]
        cnt = agg[:, 19:20]
        deg = jnp.maximum(cnt, 1.0)
        aggh = agg[:, 0:16] / deg
        aggp = agg[:, 16:19] / deg
        tab = tab_r[...]
        h = tab[:, 0:16]
        p = tab[:, 16:19]
        hn = h + jnp.dot(h, wr_r[...]) + aggh + br_r[...]
        pn = p + aggp
        z13 = jnp.zeros((tn, 13), jnp.float32)
        tabo_r[...] = jnp.concatenate([hn, pn, z13], axis=1)
        posdo_r[...] = jnp.concatenate([pn, z13], axis=1)

    return pl.pallas_call(
        body,
        grid=grid,
        in_specs=[
            pl.BlockSpec((tn, 32), lambda i: (i, 0)),
            pl.BlockSpec((2, tn, 32), lambda i: (0, i, 0)),
            pl.BlockSpec(Wr_l.shape, lambda i: (0, 0)),
            pl.BlockSpec(br_l.shape, lambda i: (0, 0)),
        ],
        out_specs=[
            pl.BlockSpec((tn, 32), lambda i: (i, 0)),
            pl.BlockSpec((tn, 16), lambda i: (i, 0)),
        ],
        out_shape=[
            jax.ShapeDtypeStruct((n, 32), jnp.float32),
            jax.ShapeDtypeStruct((n, 16), jnp.float32),
        ],
        interpret=_INTERPRET,
    )(table, agg2, Wr_l, br_l)


def _tc_set2set(table, batch2, W_ih, W_hh, b_lstm_r, Wo1, bo1r, Wo2, bo2r,
                bgraph, msteps):
    n = table.shape[0]
    hdim = 16

    def body(tab_r, bat_r, wih_r, whh_r, bl_r, wo1_r, bo1_r, wo2_r, bo2_r,
             out_r):
        h = tab_r[...][:, 0:hdim]
        bat = bat_r[...]
        ids = lax.broadcasted_iota(jnp.int32, (1, bgraph), 1)
        pm = bat == ids
        qstar = jnp.zeros((bgraph, 2 * hdim), jnp.float32)
        hs = jnp.zeros((bgraph, hdim), jnp.float32)
        cs = jnp.zeros((bgraph, hdim), jnp.float32)
        for _ in range(msteps):
            gates = (jnp.dot(qstar, wih_r[...]) + jnp.dot(hs, whh_r[...])
                     + bl_r[...])
            gi = gates[:, 0:hdim]
            gf = gates[:, hdim:2 * hdim]
            gg = gates[:, 2 * hdim:3 * hdim]
            go = gates[:, 3 * hdim:4 * hdim]
            cs = _sigmoid(gf) * cs + _sigmoid(gi) * jnp.tanh(gg)
            hs = _sigmoid(go) * jnp.tanh(cs)
            s = lax.dot_general(h, hs, (((1,), (1,)), ((), ())))
            masked = jnp.where(pm, s, -jnp.inf)
            emax = jnp.max(masked, axis=0, keepdims=True)
            emax = jnp.where(emax > -jnp.inf, emax, 0.0)
            a = jnp.exp(masked - emax)
            asum = jnp.sum(a, axis=0, keepdims=True)
            asum = jnp.where(asum > 0.0, asum, 1.0)
            an = a / asum
            r = lax.dot_general(an, h, (((0,), (0,)), ((), ())))
            qstar = jnp.concatenate([hs, r], axis=1)
        t = jnp.dot(qstar, wo1_r[...]) + bo1_r[...]
        sil = t * _sigmoid(t)
        out_r[...] = jnp.dot(sil, wo2_r[...]) + bo2_r[...]

    return pl.pallas_call(
        body,
        out_shape=jax.ShapeDtypeStruct((bgraph, 1), jnp.float32),
        interpret=_INTERPRET,
    )(table, batch2, W_ih, W_hh, b_lstm_r, Wo1, bo1r, Wo2, bo2r)


def kernel(x, edge_index, edge_attr, pos, batch, W1, b1, nW1, nb1, nW2, nb2,
           Wr, br, Wc, W_ih, W_hh, b_lstm, Wo1, bo1, Wo2, bo2):
    n, din = x.shape
    e = edge_index.shape[1]
    dedge = edge_attr.shape[1]
    hdim = W1.shape[1]
    nlayers = Wr.shape[0]
    bgraph = 64
    msteps = 3
    tn = 2000

    epad = _cdiv(e, _NW * _CH) * (_NW * _CH)
    nacc = n + 16

    src = edge_index[0].astype(jnp.int32)
    dst = edge_index[1].astype(jnp.int32)
    srcp = jnp.concatenate([src, jnp.zeros((epad - e,), jnp.int32)])
    dstp = jnp.concatenate([dst, jnp.full((epad - e,), n, jnp.int32)])
    eap = jnp.concatenate(
        [edge_attr, jnp.zeros((epad - e, dedge), jnp.float32)], axis=0)
    zinit = jnp.zeros((nacc, 32), jnp.float32)

    eye = jnp.eye(hdim, dtype=jnp.float32)
    tile_m = jnp.tile(eye, (1, hdim))
    sel_m = jnp.repeat(eye, hdim, axis=0)

    nW1a = nW1[:dedge]
    nW1b = nW1[dedge:dedge + 1]
    nb1r = nb1.reshape(1, -1)
    nb2d = nb2.reshape(hdim, hdim).T

    table, posd = _tc_init(x, pos, W1, b1.reshape(1, -1), tn)
    ea1 = _tc_edgepre(eap, nW1a, nb1r, 2048)
    for l in range(nlayers):
        gsrc, gdst = _sc_gather(table, posd, srcp, dstp)
        payload = _tc_edge(gsrc, gdst, ea1, nW1b, nW2, nb2d,
                           tile_m, sel_m, Wc[l], 4096)
        agg2 = _sc_scatter(payload, dstp, zinit)
        table, posd = _tc_update(table, agg2, Wr[l], br[l].reshape(1, -1), tn)

    out = _tc_set2set(table, batch.reshape(-1, 1).astype(jnp.int32),
                      W_ih, W_hh, b_lstm.reshape(1, -1), Wo1,
                      bo1.reshape(1, -1), Wo2, bo2.reshape(1, 1),
                      bgraph, msteps)
    return out.reshape(-1)


# speedup vs baseline: 1.3113x; 1.1230x over previous
import functools

import jax
import jax.numpy as jnp
from jax import lax
from jax.experimental import pallas as pl
from jax.experimental.pallas import tpu as pltpu
from jax.experimental.pallas import tpu_sc as plsc

_NC = 2
_NS = 16
_NW = _NC * _NS
_CH = 128

_INTERPRET = False


def _cdiv(a, b):
    return (a + b - 1) // b


_GDEPTH = 10


def _sc_gather(table, posd, srcp, dstp):
    epad = srcp.shape[0]
    per_w = epad // _NW
    nchunks = per_w // _CH
    src2 = srcp.reshape(-1, _CH)
    dst2 = dstp.reshape(-1, _CH)
    mesh = plsc.VectorSubcoreMesh(core_axis_name="c", subcore_axis_name="s")

    def body(table_h, posd_h, src_h, dst_h, gsrc_h, gdst_h,
             idx_s, idx_d, bufs, bufd, gsem, wsem):
        wid = lax.axis_index("s") * _NC + lax.axis_index("c")
        base = wid * per_w
        crow = wid * nchunks
        pltpu.sync_copy(src_h.at[pl.ds(crow, nchunks)], idx_s)
        pltpu.sync_copy(dst_h.at[pl.ds(crow, nchunks)], idx_d)
        lag = _GDEPTH // 2
        gd = [None] * nchunks
        wd = [None] * nchunks

        def fire_write(k):
            s = k % _GDEPTH
            gd[k][0].wait()
            gd[k][1].wait()
            off = base + k * _CH
            wd[k] = (
                pltpu.async_copy(bufs.at[s], gsrc_h.at[pl.ds(off, _CH)],
                                 wsem),
                pltpu.async_copy(bufd.at[s], gdst_h.at[pl.ds(off, _CH)],
                                 wsem),
            )

        for j in range(nchunks):
            s = j % _GDEPTH
            if j >= _GDEPTH:
                wd[j - _GDEPTH][0].wait()
                wd[j - _GDEPTH][1].wait()
            gd[j] = (
                pltpu.async_copy(table_h.at[idx_s.at[j]], bufs.at[s], gsem),
                pltpu.async_copy(posd_h.at[idx_d.at[j]], bufd.at[s], gsem),
            )
            if j >= lag:
                fire_write(j - lag)
        for k in range(nchunks - lag, nchunks):
            fire_write(k)
        for k in range(max(0, nchunks - _GDEPTH), nchunks):
            wd[k][0].wait()
            wd[k][1].wait()

    f = pl.kernel(
        body,
        out_type=(jax.ShapeDtypeStruct((epad, 32), jnp.float32),
                  jax.ShapeDtypeStruct((epad, 16), jnp.float32)),
        mesh=mesh,
        scratch_types=[
            pltpu.VMEM((nchunks, _CH), jnp.int32),
            pltpu.VMEM((nchunks, _CH), jnp.int32),
            pltpu.VMEM((_GDEPTH, _CH, 32), jnp.float32),
            pltpu.VMEM((_GDEPTH, _CH, 16), jnp.float32),
            pltpu.SemaphoreType.DMA,
            pltpu.SemaphoreType.DMA,
        ],
        compiler_params=pltpu.CompilerParams(use_tc_tiling_on_sc=False),
        interpret=_INTERPRET,
    )
    return f(table, posd, src2, dst2)


def _sc_scatter(payload, dstp, zinit):
    epad = payload.shape[0]
    nacc = zinit.shape[0]
    per_w = epad // _NW
    nchunks = per_w // _CH
    rpt = nacc // _NS
    mesh = plsc.VectorSubcoreMesh(core_axis_name="c", subcore_axis_name="s")

    dst2 = dstp.reshape(-1, _CH)

    def body(pay_h, dst_h, z_h, out_h, idx_v, pay_v, psem, asem, accum):
        cid = lax.axis_index("c")
        sid = lax.axis_index("s")
        wid = sid * _NC + cid
        base = wid * per_w
        pltpu.sync_copy(z_h.at[pl.ds(sid * rpt, rpt)],
                        accum.at[pl.ds(sid * rpt, rpt)])
        pltpu.sync_copy(dst_h.at[pl.ds(wid * nchunks, nchunks)], idx_v)
        plsc.subcore_barrier()
        nbuf = 4
        ld = [None] * nchunks
        ad = [None] * nchunks
        for i in range(nchunks):
            s = i % nbuf
            if i >= nbuf:
                ad[i - nbuf].wait()
            ld[i] = pltpu.async_copy(
                pay_h.at[pl.ds(base + i * _CH, _CH)], pay_v.at[s], psem)
            if i >= nbuf // 2:
                k = i - nbuf // 2
                ld[k].wait()
                ad[k] = pltpu.async_copy(
                    pay_v.at[k % nbuf], accum.at[idx_v.at[k]], asem,
                    add=True)
        for k in range(nchunks - nbuf // 2, nchunks):
            ld[k].wait()
            ad[k] = pltpu.async_copy(
                pay_v.at[k % nbuf], accum.at[idx_v.at[k]], asem, add=True)
        for k in range(max(0, nchunks - nbuf), nchunks):
            ad[k].wait()
        plsc.subcore_barrier()
        pltpu.sync_copy(accum.at[pl.ds(sid * rpt, rpt)],
                        out_h.at[cid, pl.ds(sid * rpt, rpt)])

    f = pl.kernel(
        body,
        out_type=jax.ShapeDtypeStruct((_NC, nacc, 32), jnp.float32),
        mesh=mesh,
        scratch_types=[
            pltpu.VMEM((nchunks, _CH), jnp.int32),
            pltpu.VMEM((4, _CH, 32), jnp.float32),
            pltpu.SemaphoreType.DMA,
            pltpu.SemaphoreType.DMA,
            pltpu.VMEM_SHARED((nacc, 32), jnp.float32),
        ],
        compiler_params=pltpu.CompilerParams(use_tc_tiling_on_sc=False),
        interpret=_INTERPRET,
    )
    return f(payload, dst2, zinit)


def _sigmoid(v):
    return 1.0 / (1.0 + jnp.exp(-v))


def _tc_init(x, pos, W1, b1r, tn):
    n = x.shape[0]
    din = x.shape[1]
    grid = (n // tn,)

    def body(x_r, p_r, w_r, b_r, tab_r, posd_r):
        h = jnp.dot(x_r[...], w_r[...]) + b_r[...]
        p = p_r[...]
        z13 = jnp.zeros((tn, 13), jnp.float32)
        tab_r[...] = jnp.concatenate([h, p, z13], axis=1)
        posd_r[...] = jnp.concatenate([p, z13], axis=1)

    return pl.pallas_call(
        body,
        grid=grid,
        in_specs=[
            pl.BlockSpec((tn, din), lambda i: (i, 0)),
            pl.BlockSpec((tn, 3), lambda i: (i, 0)),
            pl.BlockSpec(W1.shape, lambda i: (0, 0)),
            pl.BlockSpec(b1r.shape, lambda i: (0, 0)),
        ],
        out_specs=[
            pl.BlockSpec((tn, 32), lambda i: (i, 0)),
            pl.BlockSpec((tn, 16), lambda i: (i, 0)),
        ],
        out_shape=[
            jax.ShapeDtypeStruct((n, 32), jnp.float32),
            jax.ShapeDtypeStruct((n, 16), jnp.float32),
        ],
        interpret=_INTERPRET,
    )(x, pos, W1, b1r)


def _edge_math(hs, rel, ea1, w1b, w2, b2d, tl, sl, wc, teb):
    dist = jnp.sqrt(jnp.sum(rel * rel, axis=1, keepdims=True) + 1e-12)
    zpre = ea1 + dist * w1b
    z = zpre * _sigmoid(zpre)
    wef = jnp.dot(z, w2)
    hst = jnp.dot(hs, tl)
    msg = jnp.dot(wef * hst, sl) + jnp.dot(hs, b2d)
    cw = jnp.dot(msg, wc)
    wmsg = rel * cw
    ones = jnp.ones((teb, 1), jnp.float32)
    pad = jnp.zeros((teb, 12), jnp.float32)
    return jnp.concatenate([msg, wmsg, ones, pad], axis=1)


def _tc_edge_first(gsrc, gdst, eap, nW1a, nb1r, nW1b, nW2, nb2d, tile_m,
                   sel_m, wc_l, teb):
    epad = gsrc.shape[0]
    dedge = eap.shape[1]
    grid = (epad // teb,)

    def body(gs_r, gd_r, ea_r, w1a_r, b1_r, w1b_r, w2_r, b2d_r, tl_r, sl_r,
             wc_r, out_r, ea1_r):
        gs = gs_r[...]
        ea1 = jnp.dot(ea_r[...], w1a_r[...]) + b1_r[...]
        ea1_r[...] = ea1
        rel = gs[:, 16:19] - gd_r[...][:, 0:3]
        out_r[...] = _edge_math(gs[:, 0:16], rel, ea1, w1b_r[...], w2_r[...],
                                b2d_r[...], tl_r[...], sl_r[...], wc_r[...],
                                teb)

    return pl.pallas_call(
        body,
        grid=grid,
        in_specs=[
            pl.BlockSpec((teb, 32), lambda i: (i, 0)),
            pl.BlockSpec((teb, 16), lambda i: (i, 0)),
            pl.BlockSpec((teb, dedge), lambda i: (i, 0)),
            pl.BlockSpec(nW1a.shape, lambda i: (0, 0)),
            pl.BlockSpec(nb1r.shape, lambda i: (0, 0)),
            pl.BlockSpec(nW1b.shape, lambda i: (0, 0)),
            pl.BlockSpec(nW2.shape, lambda i: (0, 0)),
            pl.BlockSpec(nb2d.shape, lambda i: (0, 0)),
            pl.BlockSpec(tile_m.shape, lambda i: (0, 0)),
            pl.BlockSpec(sel_m.shape, lambda i: (0, 0)),
            pl.BlockSpec(wc_l.shape, lambda i: (0, 0)),
        ],
        out_specs=[
            pl.BlockSpec((teb, 32), lambda i: (i, 0)),
            pl.BlockSpec((teb, 32), lambda i: (i, 0)),
        ],
        out_shape=[
            jax.ShapeDtypeStruct((epad, 32), jnp.float32),
            jax.ShapeDtypeStruct((epad, 32), jnp.float32),
        ],
        compiler_params=pltpu.CompilerParams(
            dimension_semantics=("parallel",)),
        interpret=_INTERPRET,
    )(gsrc, gdst, eap, nW1a, nb1r, nW1b, nW2, nb2d, tile_m, sel_m, wc_l)


def _tc_edge(gsrc, gdst, ea1, nW1b, nW2, nb2d, tile_m, sel_m, wc_l, teb):
    epad = gsrc.shape[0]
    grid = (epad // teb,)

    def body(gs_r, gd_r, ea_r, w1b_r, w2_r, b2d_r, tl_r, sl_r,
             wc_r, out_r):
        gs = gs_r[...]
        rel = gs[:, 16:19] - gd_r[...][:, 0:3]
        out_r[...] = _edge_math(gs[:, 0:16], rel, ea_r[...], w1b_r[...],
                                w2_r[...], b2d_r[...], tl_r[...], sl_r[...],
                                wc_r[...], teb)

    return pl.pallas_call(
        body,
        grid=grid,
        in_specs=[
            pl.BlockSpec((teb, 32), lambda i: (i, 0)),
            pl.BlockSpec((teb, 16), lambda i: (i, 0)),
            pl.BlockSpec((teb, 32), lambda i: (i, 0)),
            pl.BlockSpec(nW1b.shape, lambda i: (0, 0)),
            pl.BlockSpec(nW2.shape, lambda i: (0, 0)),
            pl.BlockSpec(nb2d.shape, lambda i: (0, 0)),
            pl.BlockSpec(tile_m.shape, lambda i: (0, 0)),
            pl.BlockSpec(sel_m.shape, lambda i: (0, 0)),
            pl.BlockSpec(wc_l.shape, lambda i: (0, 0)),
        ],
        out_specs=pl.BlockSpec((teb, 32), lambda i: (i, 0)),
        out_shape=jax.ShapeDtypeStruct((epad, 32), jnp.float32),
        compiler_params=pltpu.CompilerParams(
            dimension_semantics=("parallel",)),
        interpret=_INTERPRET,
    )(gsrc, gdst, ea1, nW1b, nW2, nb2d, tile_m, sel_m, wc_l)


def _tc_update(table, agg2, Wr_l, br_l, tn):
    n = table.shape[0]
    nacc = agg2.shape[1]
    grid = (n // tn,)

    def body(tab_r, agg_r, wr_r, br_r, tabo_r, posdo_r):
        agg = agg_r[0] + agg_r[1]
        cnt = agg[:, 19:20]
        deg = jnp.maximum(cnt, 1.0)
        aggh = agg[:, 0:16] / deg
        aggp = agg[:, 16:19] / deg
        tab = tab_r[...]
        h = tab[:, 0:16]
        p = tab[:, 16:19]
        hn = h + jnp.dot(h, wr_r[...]) + aggh + br_r[...]
        pn = p + aggp
        z13 = jnp.zeros((tn, 13), jnp.float32)
        tabo_r[...] = jnp.concatenate([hn, pn, z13], axis=1)
        posdo_r[...] = jnp.concatenate([pn, z13], axis=1)

    return pl.pallas_call(
        body,
        grid=grid,
        in_specs=[
            pl.BlockSpec((tn, 32), lambda i: (i, 0)),
            pl.BlockSpec((2, tn, 32), lambda i: (0, i, 0)),
            pl.BlockSpec(Wr_l.shape, lambda i: (0, 0)),
            pl.BlockSpec(br_l.shape, lambda i: (0, 0)),
        ],
        out_specs=[
            pl.BlockSpec((tn, 32), lambda i: (i, 0)),
            pl.BlockSpec((tn, 16), lambda i: (i, 0)),
        ],
        out_shape=[
            jax.ShapeDtypeStruct((n, 32), jnp.float32),
            jax.ShapeDtypeStruct((n, 16), jnp.float32),
        ],
        interpret=_INTERPRET,
    )(table, agg2, Wr_l, br_l)


def _tc_set2set(table, batch2, W_ih, W_hh, b_lstm_r, Wo1, bo1r, Wo2, bo2r,
                bgraph, msteps):
    n = table.shape[0]
    hdim = 16

    def body(tab_r, bat_r, wih_r, whh_r, bl_r, wo1_r, bo1_r, wo2_r, bo2_r,
             out_r):
        h = tab_r[...][:, 0:hdim]
        bat = bat_r[...]
        ids = lax.broadcasted_iota(jnp.int32, (1, bgraph), 1)
        pm = bat == ids
        qstar = jnp.zeros((bgraph, 2 * hdim), jnp.float32)
        hs = jnp.zeros((bgraph, hdim), jnp.float32)
        cs = jnp.zeros((bgraph, hdim), jnp.float32)
        for _ in range(msteps):
            gates = (jnp.dot(qstar, wih_r[...]) + jnp.dot(hs, whh_r[...])
                     + bl_r[...])
            gi = gates[:, 0:hdim]
            gf = gates[:, hdim:2 * hdim]
            gg = gates[:, 2 * hdim:3 * hdim]
            go = gates[:, 3 * hdim:4 * hdim]
            cs = _sigmoid(gf) * cs + _sigmoid(gi) * jnp.tanh(gg)
            hs = _sigmoid(go) * jnp.tanh(cs)
            s = lax.dot_general(h, hs, (((1,), (1,)), ((), ())))
            masked = jnp.where(pm, s, -jnp.inf)
            emax = jnp.max(masked, axis=0, keepdims=True)
            emax = jnp.where(emax > -jnp.inf, emax, 0.0)
            a = jnp.exp(masked - emax)
            asum = jnp.sum(a, axis=0, keepdims=True)
            asum = jnp.where(asum > 0.0, asum, 1.0)
            an = a / asum
            r = lax.dot_general(an, h, (((0,), (0,)), ((), ())))
            qstar = jnp.concatenate([hs, r], axis=1)
        t = jnp.dot(qstar, wo1_r[...]) + bo1_r[...]
        sil = t * _sigmoid(t)
        out_r[...] = jnp.dot(sil, wo2_r[...]) + bo2_r[...]

    return pl.pallas_call(
        body,
        out_shape=jax.ShapeDtypeStruct((bgraph, 1), jnp.float32),
        interpret=_INTERPRET,
    )(table, batch2, W_ih, W_hh, b_lstm_r, Wo1, bo1r, Wo2, bo2r)


def kernel(x, edge_index, edge_attr, pos, batch, W1, b1, nW1, nb1, nW2, nb2,
           Wr, br, Wc, W_ih, W_hh, b_lstm, Wo1, bo1, Wo2, bo2):
    n, din = x.shape
    e = edge_index.shape[1]
    dedge = edge_attr.shape[1]
    hdim = W1.shape[1]
    nlayers = Wr.shape[0]
    bgraph = 64
    msteps = 3
    tn = 2000

    epad = _cdiv(e, _NW * _CH) * (_NW * _CH)
    nacc = n + 16

    src = edge_index[0].astype(jnp.int32)
    dst = edge_index[1].astype(jnp.int32)
    srcp = jnp.concatenate([src, jnp.zeros((epad - e,), jnp.int32)])
    dstp = jnp.concatenate([dst, jnp.full((epad - e,), n, jnp.int32)])
    eap = jnp.concatenate(
        [edge_attr, jnp.zeros((epad - e, dedge), jnp.float32)], axis=0)
    zinit = jnp.zeros((nacc, 32), jnp.float32)

    eye = jnp.eye(hdim, dtype=jnp.float32)
    tile_m = jnp.tile(eye, (1, hdim))
    sel_m = jnp.repeat(eye, hdim, axis=0)

    nW1a = nW1[:dedge]
    nW1b = nW1[dedge:dedge + 1]
    nb1r = nb1.reshape(1, -1)
    nb2d = nb2.reshape(hdim, hdim).T

    table, posd = _tc_init(x, pos, W1, b1.reshape(1, -1), tn)
    ea1 = None
    for l in range(nlayers):
        gsrc, gdst = _sc_gather(table, posd, srcp, dstp)
        if l == 0:
            payload, ea1 = _tc_edge_first(gsrc, gdst, eap, nW1a, nb1r, nW1b,
                                          nW2, nb2d, tile_m, sel_m, Wc[l],
                                          4096)
        else:
            payload = _tc_edge(gsrc, gdst, ea1, nW1b, nW2, nb2d,
                               tile_m, sel_m, Wc[l], 4096)
        agg2 = _sc_scatter(payload, dstp, zinit)
        table, posd = _tc_update(table, agg2, Wr[l], br[l].reshape(1, -1), tn)

    out = _tc_set2set(table, batch.reshape(-1, 1).astype(jnp.int32),
                      W_ih, W_hh, b_lstm.reshape(1, -1), Wo1,
                      bo1.reshape(1, -1), Wo2, bo2.reshape(1, 1),
                      bgraph, msteps)
    return out.reshape(-1)


# speedup vs baseline: 1.3114x; 1.0000x over previous
import functools

import jax
import jax.numpy as jnp
from jax import lax
from jax.experimental import pallas as pl
from jax.experimental.pallas import tpu as pltpu
from jax.experimental.pallas import tpu_sc as plsc

_NC = 2
_NS = 16
_NW = _NC * _NS
_CH = 128

_INTERPRET = False


def _cdiv(a, b):
    return (a + b - 1) // b


_GDEPTH = 10


def _sc_gather(table, posd, srcp, dstp):
    epad = srcp.shape[0]
    per_w = epad // _NW
    nchunks = per_w // _CH
    src2 = srcp.reshape(-1, _CH)
    dst2 = dstp.reshape(-1, _CH)
    mesh = plsc.VectorSubcoreMesh(core_axis_name="c", subcore_axis_name="s")

    def body(table_h, posd_h, src_h, dst_h, gsrc_h,
             idx_s, idx_d, bufs, bufd, gsem, wsem):
        wid = lax.axis_index("s") * _NC + lax.axis_index("c")
        base = wid * per_w
        crow = wid * nchunks
        pltpu.sync_copy(src_h.at[pl.ds(crow, nchunks)], idx_s)
        pltpu.sync_copy(dst_h.at[pl.ds(crow, nchunks)], idx_d)
        lag = _GDEPTH // 2
        gd = [None] * nchunks
        wd = [None] * nchunks

        def fire_write(k):
            s = k % _GDEPTH
            gd[k][0].wait()
            gd[k][1].wait()

            def rowfix(r, carry):
                a = bufs[s, r, pl.ds(16, 16)]
                b = bufd[s, r, pl.ds(0, 16)]
                bufs[s, r, pl.ds(16, 16)] = a - b
                return carry

            lax.fori_loop(0, _CH, rowfix, 0)
            off = base + k * _CH
            wd[k] = pltpu.async_copy(bufs.at[s], gsrc_h.at[pl.ds(off, _CH)],
                                     wsem)

        for j in range(nchunks):
            s = j % _GDEPTH
            if j >= _GDEPTH:
                wd[j - _GDEPTH].wait()
            gd[j] = (
                pltpu.async_copy(table_h.at[idx_s.at[j]], bufs.at[s], gsem),
                pltpu.async_copy(posd_h.at[idx_d.at[j]], bufd.at[s], gsem),
            )
            if j >= lag:
                fire_write(j - lag)
        for k in range(nchunks - lag, nchunks):
            fire_write(k)
        for k in range(max(0, nchunks - _GDEPTH), nchunks):
            wd[k].wait()

    f = pl.kernel(
        body,
        out_type=jax.ShapeDtypeStruct((epad, 32), jnp.float32),
        mesh=mesh,
        scratch_types=[
            pltpu.VMEM((nchunks, _CH), jnp.int32),
            pltpu.VMEM((nchunks, _CH), jnp.int32),
            pltpu.VMEM((_GDEPTH, _CH, 32), jnp.float32),
            pltpu.VMEM((_GDEPTH, _CH, 16), jnp.float32),
            pltpu.SemaphoreType.DMA,
            pltpu.SemaphoreType.DMA,
        ],
        compiler_params=pltpu.CompilerParams(use_tc_tiling_on_sc=False),
        interpret=_INTERPRET,
    )
    return f(table, posd, src2, dst2)


def _sc_scatter(payload, dstp, zinit):
    epad = payload.shape[0]
    nacc = zinit.shape[0]
    per_w = epad // _NW
    nchunks = per_w // _CH
    rpt = nacc // _NS
    mesh = plsc.VectorSubcoreMesh(core_axis_name="c", subcore_axis_name="s")

    dst2 = dstp.reshape(-1, _CH)

    def body(pay_h, dst_h, z_h, out_h, idx_v, pay_v, psem, asem, accum):
        cid = lax.axis_index("c")
        sid = lax.axis_index("s")
        wid = sid * _NC + cid
        base = wid * per_w
        pltpu.sync_copy(z_h.at[pl.ds(sid * rpt, rpt)],
                        accum.at[pl.ds(sid * rpt, rpt)])
        pltpu.sync_copy(dst_h.at[pl.ds(wid * nchunks, nchunks)], idx_v)
        plsc.subcore_barrier()
        nbuf = 4
        ld = [None] * nchunks
        ad = [None] * nchunks
        for i in range(nchunks):
            s = i % nbuf
            if i >= nbuf:
                ad[i - nbuf].wait()
            ld[i] = pltpu.async_copy(
                pay_h.at[pl.ds(base + i * _CH, _CH)], pay_v.at[s], psem)
            if i >= nbuf // 2:
                k = i - nbuf // 2
                ld[k].wait()
                ad[k] = pltpu.async_copy(
                    pay_v.at[k % nbuf], accum.at[idx_v.at[k]], asem,
                    add=True)
        for k in range(nchunks - nbuf // 2, nchunks):
            ld[k].wait()
            ad[k] = pltpu.async_copy(
                pay_v.at[k % nbuf], accum.at[idx_v.at[k]], asem, add=True)
        for k in range(max(0, nchunks - nbuf), nchunks):
            ad[k].wait()
        plsc.subcore_barrier()
        pltpu.sync_copy(accum.at[pl.ds(sid * rpt, rpt)],
                        out_h.at[cid, pl.ds(sid * rpt, rpt)])

    f = pl.kernel(
        body,
        out_type=jax.ShapeDtypeStruct((_NC, nacc, 32), jnp.float32),
        mesh=mesh,
        scratch_types=[
            pltpu.VMEM((nchunks, _CH), jnp.int32),
            pltpu.VMEM((4, _CH, 32), jnp.float32),
            pltpu.SemaphoreType.DMA,
            pltpu.SemaphoreType.DMA,
            pltpu.VMEM_SHARED((nacc, 32), jnp.float32),
        ],
        compiler_params=pltpu.CompilerParams(use_tc_tiling_on_sc=False),
        interpret=_INTERPRET,
    )
    return f(payload, dst2, zinit)


def _sigmoid(v):
    return 1.0 / (1.0 + jnp.exp(-v))


def _tc_init(x, pos, W1, b1r, tn):
    n = x.shape[0]
    din = x.shape[1]
    grid = (n // tn,)

    def body(x_r, p_r, w_r, b_r, tab_r, posd_r):
        h = jnp.dot(x_r[...], w_r[...]) + b_r[...]
        p = p_r[...]
        z13 = jnp.zeros((tn, 13), jnp.float32)
        tab_r[...] = jnp.concatenate([h, p, z13], axis=1)
        posd_r[...] = jnp.concatenate([p, z13], axis=1)

    return pl.pallas_call(
        body,
        grid=grid,
        in_specs=[
            pl.BlockSpec((tn, din), lambda i: (i, 0)),
            pl.BlockSpec((tn, 3), lambda i: (i, 0)),
            pl.BlockSpec(W1.shape, lambda i: (0, 0)),
            pl.BlockSpec(b1r.shape, lambda i: (0, 0)),
        ],
        out_specs=[
            pl.BlockSpec((tn, 32), lambda i: (i, 0)),
            pl.BlockSpec((tn, 16), lambda i: (i, 0)),
        ],
        out_shape=[
            jax.ShapeDtypeStruct((n, 32), jnp.float32),
            jax.ShapeDtypeStruct((n, 16), jnp.float32),
        ],
        interpret=_INTERPRET,
    )(x, pos, W1, b1r)


def _edge_math(hs, rel, ea1, w1b, w2, b2d, tl, sl, wc, teb):
    dist = jnp.sqrt(jnp.sum(rel * rel, axis=1, keepdims=True) + 1e-12)
    zpre = ea1 + dist * w1b
    z = zpre * _sigmoid(zpre)
    wef = jnp.dot(z, w2)
    hst = jnp.dot(hs, tl)
    msg = jnp.dot(wef * hst, sl) + jnp.dot(hs, b2d)
    cw = jnp.dot(msg, wc)
    wmsg = rel * cw
    ones = jnp.ones((teb, 1), jnp.float32)
    pad = jnp.zeros((teb, 12), jnp.float32)
    return jnp.concatenate([msg, wmsg, ones, pad], axis=1)


def _tc_edge_first(gsrc, eap, nW1a, nb1r, nW1b, nW2, nb2d, tile_m,
                   sel_m, wc_l, teb):
    epad = gsrc.shape[0]
    dedge = eap.shape[1]
    grid = (epad // teb,)

    def body(gs_r, ea_r, w1a_r, b1_r, w1b_r, w2_r, b2d_r, tl_r, sl_r,
             wc_r, out_r, ea1_r):
        gs = gs_r[...]
        ea1 = jnp.dot(ea_r[...], w1a_r[...]) + b1_r[...]
        ea1_r[...] = ea1
        out_r[...] = _edge_math(gs[:, 0:16], gs[:, 16:19], ea1, w1b_r[...],
                                w2_r[...], b2d_r[...], tl_r[...], sl_r[...],
                                wc_r[...], teb)

    return pl.pallas_call(
        body,
        grid=grid,
        in_specs=[
            pl.BlockSpec((teb, 32), lambda i: (i, 0)),
            pl.BlockSpec((teb, dedge), lambda i: (i, 0)),
            pl.BlockSpec(nW1a.shape, lambda i: (0, 0)),
            pl.BlockSpec(nb1r.shape, lambda i: (0, 0)),
            pl.BlockSpec(nW1b.shape, lambda i: (0, 0)),
            pl.BlockSpec(nW2.shape, lambda i: (0, 0)),
            pl.BlockSpec(nb2d.shape, lambda i: (0, 0)),
            pl.BlockSpec(tile_m.shape, lambda i: (0, 0)),
            pl.BlockSpec(sel_m.shape, lambda i: (0, 0)),
            pl.BlockSpec(wc_l.shape, lambda i: (0, 0)),
        ],
        out_specs=[
            pl.BlockSpec((teb, 32), lambda i: (i, 0)),
            pl.BlockSpec((teb, 32), lambda i: (i, 0)),
        ],
        out_shape=[
            jax.ShapeDtypeStruct((epad, 32), jnp.float32),
            jax.ShapeDtypeStruct((epad, 32), jnp.float32),
        ],
        compiler_params=pltpu.CompilerParams(
            dimension_semantics=("parallel",)),
        interpret=_INTERPRET,
    )(gsrc, eap, nW1a, nb1r, nW1b, nW2, nb2d, tile_m, sel_m, wc_l)


def _tc_edge(gsrc, ea1, nW1b, nW2, nb2d, tile_m, sel_m, wc_l, teb):
    epad = gsrc.shape[0]
    grid = (epad // teb,)

    def body(gs_r, ea_r, w1b_r, w2_r, b2d_r, tl_r, sl_r,
             wc_r, out_r):
        gs = gs_r[...]
        out_r[...] = _edge_math(gs[:, 0:16], gs[:, 16:19], ea_r[...],
                                w1b_r[...], w2_r[...], b2d_r[...], tl_r[...],
                                sl_r[...], wc_r[...], teb)

    return pl.pallas_call(
        body,
        grid=grid,
        in_specs=[
            pl.BlockSpec((teb, 32), lambda i: (i, 0)),
            pl.BlockSpec((teb, 32), lambda i: (i, 0)),
            pl.BlockSpec(nW1b.shape, lambda i: (0, 0)),
            pl.BlockSpec(nW2.shape, lambda i: (0, 0)),
            pl.BlockSpec(nb2d.shape, lambda i: (0, 0)),
            pl.BlockSpec(tile_m.shape, lambda i: (0, 0)),
            pl.BlockSpec(sel_m.shape, lambda i: (0, 0)),
            pl.BlockSpec(wc_l.shape, lambda i: (0, 0)),
        ],
        out_specs=pl.BlockSpec((teb, 32), lambda i: (i, 0)),
        out_shape=jax.ShapeDtypeStruct((epad, 32), jnp.float32),
        compiler_params=pltpu.CompilerParams(
            dimension_semantics=("parallel",)),
        interpret=_INTERPRET,
    )(gsrc, ea1, nW1b, nW2, nb2d, tile_m, sel_m, wc_l)


def _tc_update(table, agg2, Wr_l, br_l, tn):
    n = table.shape[0]
    nacc = agg2.shape[1]
    grid = (n // tn,)

    def body(tab_r, agg_r, wr_r, br_r, tabo_r, posdo_r):
        agg = agg_r[0] + agg_r[1]
        cnt = agg[:, 19:20]
        deg = jnp.maximum(cnt, 1.0)
        aggh = agg[:, 0:16] / deg
        aggp = agg[:, 16:19] / deg
        tab = tab_r[...]
        h = tab[:, 0:16]
        p = tab[:, 16:19]
        hn = h + jnp.dot(h, wr_r[...]) + aggh + br_r[...]
        pn = p + aggp
        z13 = jnp.zeros((tn, 13), jnp.float32)
        tabo_r[...] = jnp.concatenate([hn, pn, z13], axis=1)
        posdo_r[...] = jnp.concatenate([pn, z13], axis=1)

    return pl.pallas_call(
        body,
        grid=grid,
        in_specs=[
            pl.BlockSpec((tn, 32), lambda i: (i, 0)),
            pl.BlockSpec((2, tn, 32), lambda i: (0, i, 0)),
            pl.BlockSpec(Wr_l.shape, lambda i: (0, 0)),
            pl.BlockSpec(br_l.shape, lambda i: (0, 0)),
        ],
        out_specs=[
            pl.BlockSpec((tn, 32), lambda i: (i, 0)),
            pl.BlockSpec((tn, 16), lambda i: (i, 0)),
        ],
        out_shape=[
            jax.ShapeDtypeStruct((n, 32), jnp.float32),
            jax.ShapeDtypeStruct((n, 16), jnp.float32),
        ],
        interpret=_INTERPRET,
    )(table, agg2, Wr_l, br_l)


def _tc_set2set(table, batch2, W_ih, W_hh, b_lstm_r, Wo1, bo1r, Wo2, bo2r,
                bgraph, msteps):
    n = table.shape[0]
    hdim = 16

    def body(tab_r, bat_r, wih_r, whh_r, bl_r, wo1_r, bo1_r, wo2_r, bo2_r,
             out_r):
        h = tab_r[...][:, 0:hdim]
        bat = bat_r[...]
        ids = lax.broadcasted_iota(jnp.int32, (1, bgraph), 1)
        pm = bat == ids
        qstar = jnp.zeros((bgraph, 2 * hdim), jnp.float32)
        hs = jnp.zeros((bgraph, hdim), jnp.float32)
        cs = jnp.zeros((bgraph, hdim), jnp.float32)
        for _ in range(msteps):
            gates = (jnp.dot(qstar, wih_r[...]) + jnp.dot(hs, whh_r[...])
                     + bl_r[...])
            gi = gates[:, 0:hdim]
            gf = gates[:, hdim:2 * hdim]
            gg = gates[:, 2 * hdim:3 * hdim]
            go = gates[:, 3 * hdim:4 * hdim]
            cs = _sigmoid(gf) * cs + _sigmoid(gi) * jnp.tanh(gg)
            hs = _sigmoid(go) * jnp.tanh(cs)
            s = lax.dot_general(h, hs, (((1,), (1,)), ((), ())))
            masked = jnp.where(pm, s, -jnp.inf)
            emax = jnp.max(masked, axis=0, keepdims=True)
            emax = jnp.where(emax > -jnp.inf, emax, 0.0)
            a = jnp.exp(masked - emax)
            asum = jnp.sum(a, axis=0, keepdims=True)
            asum = jnp.where(asum > 0.0, asum, 1.0)
            an = a / asum
            r = lax.dot_general(an, h, (((0,), (0,)), ((), ())))
            qstar = jnp.concatenate([hs, r], axis=1)
        t = jnp.dot(qstar, wo1_r[...]) + bo1_r[...]
        sil = t * _sigmoid(t)
        out_r[...] = jnp.dot(sil, wo2_r[...]) + bo2_r[...]

    return pl.pallas_call(
        body,
        out_shape=jax.ShapeDtypeStruct((bgraph, 1), jnp.float32),
        interpret=_INTERPRET,
    )(table, batch2, W_ih, W_hh, b_lstm_r, Wo1, bo1r, Wo2, bo2r)


def kernel(x, edge_index, edge_attr, pos, batch, W1, b1, nW1, nb1, nW2, nb2,
           Wr, br, Wc, W_ih, W_hh, b_lstm, Wo1, bo1, Wo2, bo2):
    n, din = x.shape
    e = edge_index.shape[1]
    dedge = edge_attr.shape[1]
    hdim = W1.shape[1]
    nlayers = Wr.shape[0]
    bgraph = 64
    msteps = 3
    tn = 2000

    epad = _cdiv(e, _NW * _CH) * (_NW * _CH)
    nacc = n + 16

    src = edge_index[0].astype(jnp.int32)
    dst = edge_index[1].astype(jnp.int32)
    srcp = jnp.concatenate([src, jnp.zeros((epad - e,), jnp.int32)])
    dstp = jnp.concatenate([dst, jnp.full((epad - e,), n, jnp.int32)])
    eap = jnp.concatenate(
        [edge_attr, jnp.zeros((epad - e, dedge), jnp.float32)], axis=0)
    zinit = jnp.zeros((nacc, 32), jnp.float32)

    eye = jnp.eye(hdim, dtype=jnp.float32)
    tile_m = jnp.tile(eye, (1, hdim))
    sel_m = jnp.repeat(eye, hdim, axis=0)

    nW1a = nW1[:dedge]
    nW1b = nW1[dedge:dedge + 1]
    nb1r = nb1.reshape(1, -1)
    nb2d = nb2.reshape(hdim, hdim).T

    table, posd = _tc_init(x, pos, W1, b1.reshape(1, -1), tn)
    ea1 = None
    for l in range(nlayers):
        gsrc = _sc_gather(table, posd, srcp, dstp)
        if l == 0:
            payload, ea1 = _tc_edge_first(gsrc, eap, nW1a, nb1r, nW1b,
                                          nW2, nb2d, tile_m, sel_m, Wc[l],
                                          4096)
        else:
            payload = _tc_edge(gsrc, ea1, nW1b, nW2, nb2d,
                               tile_m, sel_m, Wc[l], 4096)
        agg2 = _sc_scatter(payload, dstp, zinit)
        table, posd = _tc_update(table, agg2, Wr[l], br[l].reshape(1, -1), tn)

    out = _tc_set2set(table, batch.reshape(-1, 1).astype(jnp.int32),
                      W_ih, W_hh, b_lstm.reshape(1, -1), Wo1,
                      bo1.reshape(1, -1), Wo2, bo2.reshape(1, 1),
                      bgraph, msteps)
    return out.reshape(-1)


# speedup vs baseline: 1.3122x; 1.0006x over previous
import functools

import jax
import jax.numpy as jnp
from jax import lax
from jax.experimental import pallas as pl
from jax.experimental.pallas import tpu as pltpu
from jax.experimental.pallas import tpu_sc as plsc

_NC = 2
_NS = 16
_NW = _NC * _NS
_CH = 128

_INTERPRET = False


def _cdiv(a, b):
    return (a + b - 1) // b


_GDEPTH = 14


def _sc_gather(table, posd, srcp, dstp):
    epad = srcp.shape[0]
    per_w = epad // _NW
    nchunks = per_w // _CH
    src2 = srcp.reshape(-1, _CH)
    dst2 = dstp.reshape(-1, _CH)
    mesh = plsc.VectorSubcoreMesh(core_axis_name="c", subcore_axis_name="s")

    def body(table_h, posd_h, src_h, dst_h, gsrc_h,
             idx_s, idx_d, bufs, bufd, gsem, wsem):
        wid = lax.axis_index("s") * _NC + lax.axis_index("c")
        base = wid * per_w
        crow = wid * nchunks
        pltpu.sync_copy(src_h.at[pl.ds(crow, nchunks)], idx_s)
        pltpu.sync_copy(dst_h.at[pl.ds(crow, nchunks)], idx_d)
        lag = _GDEPTH // 2
        gd = [None] * nchunks
        wd = [None] * nchunks

        def fire_write(k):
            s = k % _GDEPTH
            gd[k][0].wait()
            gd[k][1].wait()

            def rowfix(r, carry):
                a = bufs[s, r, pl.ds(16, 16)]
                b = bufd[s, r, pl.ds(0, 16)]
                bufs[s, r, pl.ds(16, 16)] = a - b
                return carry

            lax.fori_loop(0, _CH, rowfix, 0)
            off = base + k * _CH
            wd[k] = pltpu.async_copy(bufs.at[s], gsrc_h.at[pl.ds(off, _CH)],
                                     wsem)

        for j in range(nchunks):
            s = j % _GDEPTH
            if j >= _GDEPTH:
                wd[j - _GDEPTH].wait()
            gd[j] = (
                pltpu.async_copy(table_h.at[idx_s.at[j]], bufs.at[s], gsem),
                pltpu.async_copy(posd_h.at[idx_d.at[j]], bufd.at[s], gsem),
            )
            if j >= lag:
                fire_write(j - lag)
        for k in range(nchunks - lag, nchunks):
            fire_write(k)
        for k in range(max(0, nchunks - _GDEPTH), nchunks):
            wd[k].wait()

    f = pl.kernel(
        body,
        out_type=jax.ShapeDtypeStruct((epad, 32), jnp.float32),
        mesh=mesh,
        scratch_types=[
            pltpu.VMEM((nchunks, _CH), jnp.int32),
            pltpu.VMEM((nchunks, _CH), jnp.int32),
            pltpu.VMEM((_GDEPTH, _CH, 32), jnp.float32),
            pltpu.VMEM((_GDEPTH, _CH, 16), jnp.float32),
            pltpu.SemaphoreType.DMA,
            pltpu.SemaphoreType.DMA,
        ],
        compiler_params=pltpu.CompilerParams(use_tc_tiling_on_sc=False),
        interpret=_INTERPRET,
    )
    return f(table, posd, src2, dst2)


def _sc_scatter(payload, dstp, zinit):
    epad = payload.shape[0]
    nacc = zinit.shape[0]
    per_w = epad // _NW
    nchunks = per_w // _CH
    rpt = nacc // _NS
    mesh = plsc.VectorSubcoreMesh(core_axis_name="c", subcore_axis_name="s")

    dst2 = dstp.reshape(-1, _CH)

    def body(pay_h, dst_h, z_h, out_h, idx_v, pay_v, psem, asem, accum):
        cid = lax.axis_index("c")
        sid = lax.axis_index("s")
        wid = sid * _NC + cid
        base = wid * per_w
        pltpu.sync_copy(z_h.at[pl.ds(sid * rpt, rpt)],
                        accum.at[pl.ds(sid * rpt, rpt)])
        pltpu.sync_copy(dst_h.at[pl.ds(wid * nchunks, nchunks)], idx_v)
        plsc.subcore_barrier()
        nbuf = 6
        ld = [None] * nchunks
        ad = [None] * nchunks
        for i in range(nchunks):
            s = i % nbuf
            if i >= nbuf:
                ad[i - nbuf].wait()
            ld[i] = pltpu.async_copy(
                pay_h.at[pl.ds(base + i * _CH, _CH)], pay_v.at[s], psem)
            if i >= nbuf // 2:
                k = i - nbuf // 2
                ld[k].wait()
                ad[k] = pltpu.async_copy(
                    pay_v.at[k % nbuf], accum.at[idx_v.at[k]], asem,
                    add=True)
        for k in range(nchunks - nbuf // 2, nchunks):
            ld[k].wait()
            ad[k] = pltpu.async_copy(
                pay_v.at[k % nbuf], accum.at[idx_v.at[k]], asem, add=True)
        for k in range(max(0, nchunks - nbuf), nchunks):
            ad[k].wait()
        plsc.subcore_barrier()
        pltpu.sync_copy(accum.at[pl.ds(sid * rpt, rpt)],
                        out_h.at[cid, pl.ds(sid * rpt, rpt)])

    f = pl.kernel(
        body,
        out_type=jax.ShapeDtypeStruct((_NC, nacc, 32), jnp.float32),
        mesh=mesh,
        scratch_types=[
            pltpu.VMEM((nchunks, _CH), jnp.int32),
            pltpu.VMEM((6, _CH, 32), jnp.float32),
            pltpu.SemaphoreType.DMA,
            pltpu.SemaphoreType.DMA,
            pltpu.VMEM_SHARED((nacc, 32), jnp.float32),
        ],
        compiler_params=pltpu.CompilerParams(use_tc_tiling_on_sc=False),
        interpret=_INTERPRET,
    )
    return f(payload, dst2, zinit)


def _sigmoid(v):
    return 1.0 / (1.0 + jnp.exp(-v))


def _tc_init(x, pos, W1, b1r, tn):
    n = x.shape[0]
    din = x.shape[1]
    grid = (n // tn,)

    def body(x_r, p_r, w_r, b_r, tab_r, posd_r):
        h = jnp.dot(x_r[...], w_r[...]) + b_r[...]
        p = p_r[...]
        z13 = jnp.zeros((tn, 13), jnp.float32)
        tab_r[...] = jnp.concatenate([h, p, z13], axis=1)
        posd_r[...] = jnp.concatenate([p, z13], axis=1)

    return pl.pallas_call(
        body,
        grid=grid,
        in_specs=[
            pl.BlockSpec((tn, din), lambda i: (i, 0)),
            pl.BlockSpec((tn, 3), lambda i: (i, 0)),
            pl.BlockSpec(W1.shape, lambda i: (0, 0)),
            pl.BlockSpec(b1r.shape, lambda i: (0, 0)),
        ],
        out_specs=[
            pl.BlockSpec((tn, 32), lambda i: (i, 0)),
            pl.BlockSpec((tn, 16), lambda i: (i, 0)),
        ],
        out_shape=[
            jax.ShapeDtypeStruct((n, 32), jnp.float32),
            jax.ShapeDtypeStruct((n, 16), jnp.float32),
        ],
        interpret=_INTERPRET,
    )(x, pos, W1, b1r)


def _edge_math(hs, rel, ea1, w1b, w2, b2d, tl, sl, wc, teb):
    dist = jnp.sqrt(jnp.sum(rel * rel, axis=1, keepdims=True) + 1e-12)
    zpre = ea1 + dist * w1b
    z = zpre * _sigmoid(zpre)
    wef = jnp.dot(z, w2)
    hst = jnp.dot(hs, tl)
    msg = jnp.dot(wef * hst, sl) + jnp.dot(hs, b2d)
    cw = jnp.dot(msg, wc)
    wmsg = rel * cw
    ones = jnp.ones((teb, 1), jnp.float32)
    pad = jnp.zeros((teb, 12), jnp.float32)
    return jnp.concatenate([msg, wmsg, ones, pad], axis=1)


def _tc_edge_first(gsrc, eap, nW1a, nb1r, nW1b, nW2, nb2d, tile_m,
                   sel_m, wc_l, teb):
    epad = gsrc.shape[0]
    dedge = eap.shape[1]
    grid = (epad // teb,)

    def body(gs_r, ea_r, w1a_r, b1_r, w1b_r, w2_r, b2d_r, tl_r, sl_r,
             wc_r, out_r, ea1_r):
        gs = gs_r[...]
        ea1 = jnp.dot(ea_r[...], w1a_r[...]) + b1_r[...]
        ea1_r[...] = ea1
        out_r[...] = _edge_math(gs[:, 0:16], gs[:, 16:19], ea1, w1b_r[...],
                                w2_r[...], b2d_r[...], tl_r[...], sl_r[...],
                                wc_r[...], teb)

    return pl.pallas_call(
        body,
        grid=grid,
        in_specs=[
            pl.BlockSpec((teb, 32), lambda i: (i, 0)),
            pl.BlockSpec((teb, dedge), lambda i: (i, 0)),
            pl.BlockSpec(nW1a.shape, lambda i: (0, 0)),
            pl.BlockSpec(nb1r.shape, lambda i: (0, 0)),
            pl.BlockSpec(nW1b.shape, lambda i: (0, 0)),
            pl.BlockSpec(nW2.shape, lambda i: (0, 0)),
            pl.BlockSpec(nb2d.shape, lambda i: (0, 0)),
            pl.BlockSpec(tile_m.shape, lambda i: (0, 0)),
            pl.BlockSpec(sel_m.shape, lambda i: (0, 0)),
            pl.BlockSpec(wc_l.shape, lambda i: (0, 0)),
        ],
        out_specs=[
            pl.BlockSpec((teb, 32), lambda i: (i, 0)),
            pl.BlockSpec((teb, 32), lambda i: (i, 0)),
        ],
        out_shape=[
            jax.ShapeDtypeStruct((epad, 32), jnp.float32),
            jax.ShapeDtypeStruct((epad, 32), jnp.float32),
        ],
        compiler_params=pltpu.CompilerParams(
            dimension_semantics=("parallel",)),
        interpret=_INTERPRET,
    )(gsrc, eap, nW1a, nb1r, nW1b, nW2, nb2d, tile_m, sel_m, wc_l)


def _tc_edge(gsrc, ea1, nW1b, nW2, nb2d, tile_m, sel_m, wc_l, teb):
    epad = gsrc.shape[0]
    grid = (epad // teb,)

    def body(gs_r, ea_r, w1b_r, w2_r, b2d_r, tl_r, sl_r,
             wc_r, out_r):
        gs = gs_r[...]
        out_r[...] = _edge_math(gs[:, 0:16], gs[:, 16:19], ea_r[...],
                                w1b_r[...], w2_r[...], b2d_r[...], tl_r[...],
                                sl_r[...], wc_r[...], teb)

    return pl.pallas_call(
        body,
        grid=grid,
        in_specs=[
            pl.BlockSpec((teb, 32), lambda i: (i, 0)),
            pl.BlockSpec((teb, 32), lambda i: (i, 0)),
            pl.BlockSpec(nW1b.shape, lambda i: (0, 0)),
            pl.BlockSpec(nW2.shape, lambda i: (0, 0)),
            pl.BlockSpec(nb2d.shape, lambda i: (0, 0)),
            pl.BlockSpec(tile_m.shape, lambda i: (0, 0)),
            pl.BlockSpec(sel_m.shape, lambda i: (0, 0)),
            pl.BlockSpec(wc_l.shape, lambda i: (0, 0)),
        ],
        out_specs=pl.BlockSpec((teb, 32), lambda i: (i, 0)),
        out_shape=jax.ShapeDtypeStruct((epad, 32), jnp.float32),
        compiler_params=pltpu.CompilerParams(
            dimension_semantics=("parallel",)),
        interpret=_INTERPRET,
    )(gsrc, ea1, nW1b, nW2, nb2d, tile_m, sel_m, wc_l)


def _tc_update(table, agg2, Wr_l, br_l, tn):
    n = table.shape[0]
    nacc = agg2.shape[1]
    grid = (n // tn,)

    def body(tab_r, agg_r, wr_r, br_r, tabo_r, posdo_r):
        agg = agg_r[0] + agg_r[1]
        cnt = agg[:, 19:20]
        deg = jnp.maximum(cnt, 1.0)
        aggh = agg[:, 0:16] / deg
        aggp = agg[:, 16:19] / deg
        tab = tab_r[...]
        h = tab[:, 0:16]
        p = tab[:, 16:19]
        hn = h + jnp.dot(h, wr_r[...]) + aggh + br_r[...]
        pn = p + aggp
        z13 = jnp.zeros((tn, 13), jnp.float32)
        tabo_r[...] = jnp.concatenate([hn, pn, z13], axis=1)
        posdo_r[...] = jnp.concatenate([pn, z13], axis=1)

    return pl.pallas_call(
        body,
        grid=grid,
        in_specs=[
            pl.BlockSpec((tn, 32), lambda i: (i, 0)),
            pl.BlockSpec((2, tn, 32), lambda i: (0, i, 0)),
            pl.BlockSpec(Wr_l.shape, lambda i: (0, 0)),
            pl.BlockSpec(br_l.shape, lambda i: (0, 0)),
        ],
        out_specs=[
            pl.BlockSpec((tn, 32), lambda i: (i, 0)),
            pl.BlockSpec((tn, 16), lambda i: (i, 0)),
        ],
        out_shape=[
            jax.ShapeDtypeStruct((n, 32), jnp.float32),
            jax.ShapeDtypeStruct((n, 16), jnp.float32),
        ],
        interpret=_INTERPRET,
    )(table, agg2, Wr_l, br_l)


def _tc_set2set(table, batch2, W_ih, W_hh, b_lstm_r, Wo1, bo1r, Wo2, bo2r,
                bgraph, msteps):
    n = table.shape[0]
    hdim = 16

    def body(tab_r, bat_r, wih_r, whh_r, bl_r, wo1_r, bo1_r, wo2_r, bo2_r,
             out_r):
        h = tab_r[...][:, 0:hdim]
        bat = bat_r[...]
        ids = lax.broadcasted_iota(jnp.int32, (1, bgraph), 1)
        pm = bat == ids
        qstar = jnp.zeros((bgraph, 2 * hdim), jnp.float32)
        hs = jnp.zeros((bgraph, hdim), jnp.float32)
        cs = jnp.zeros((bgraph, hdim), jnp.float32)
        for _ in range(msteps):
            gates = (jnp.dot(qstar, wih_r[...]) + jnp.dot(hs, whh_r[...])
                     + bl_r[...])
            gi = gates[:, 0:hdim]
            gf = gates[:, hdim:2 * hdim]
            gg = gates[:, 2 * hdim:3 * hdim]
            go = gates[:, 3 * hdim:4 * hdim]
            cs = _sigmoid(gf) * cs + _sigmoid(gi) * jnp.tanh(gg)
            hs = _sigmoid(go) * jnp.tanh(cs)
            s = lax.dot_general(h, hs, (((1,), (1,)), ((), ())))
            masked = jnp.where(pm, s, -jnp.inf)
            emax = jnp.max(masked, axis=0, keepdims=True)
            emax = jnp.where(emax > -jnp.inf, emax, 0.0)
            a = jnp.exp(masked - emax)
            asum = jnp.sum(a, axis=0, keepdims=True)
            asum = jnp.where(asum > 0.0, asum, 1.0)
            an = a / asum
            r = lax.dot_general(an, h, (((0,), (0,)), ((), ())))
            qstar = jnp.concatenate([hs, r], axis=1)
        t = jnp.dot(qstar, wo1_r[...]) + bo1_r[...]
        sil = t * _sigmoid(t)
        out_r[...] = jnp.dot(sil, wo2_r[...]) + bo2_r[...]

    return pl.pallas_call(
        body,
        out_shape=jax.ShapeDtypeStruct((bgraph, 1), jnp.float32),
        interpret=_INTERPRET,
    )(table, batch2, W_ih, W_hh, b_lstm_r, Wo1, bo1r, Wo2, bo2r)


def kernel(x, edge_index, edge_attr, pos, batch, W1, b1, nW1, nb1, nW2, nb2,
           Wr, br, Wc, W_ih, W_hh, b_lstm, Wo1, bo1, Wo2, bo2):
    n, din = x.shape
    e = edge_index.shape[1]
    dedge = edge_attr.shape[1]
    hdim = W1.shape[1]
    nlayers = Wr.shape[0]
    bgraph = 64
    msteps = 3
    tn = 2000

    epad = _cdiv(e, _NW * _CH) * (_NW * _CH)
    nacc = n + 16

    src = edge_index[0].astype(jnp.int32)
    dst = edge_index[1].astype(jnp.int32)
    srcp = jnp.concatenate([src, jnp.zeros((epad - e,), jnp.int32)])
    dstp = jnp.concatenate([dst, jnp.full((epad - e,), n, jnp.int32)])
    eap = jnp.concatenate(
        [edge_attr, jnp.zeros((epad - e, dedge), jnp.float32)], axis=0)
    zinit = jnp.zeros((nacc, 32), jnp.float32)

    eye = jnp.eye(hdim, dtype=jnp.float32)
    tile_m = jnp.tile(eye, (1, hdim))
    sel_m = jnp.repeat(eye, hdim, axis=0)

    nW1a = nW1[:dedge]
    nW1b = nW1[dedge:dedge + 1]
    nb1r = nb1.reshape(1, -1)
    nb2d = nb2.reshape(hdim, hdim).T

    table, posd = _tc_init(x, pos, W1, b1.reshape(1, -1), tn)
    ea1 = None
    for l in range(nlayers):
        gsrc = _sc_gather(table, posd, srcp, dstp)
        if l == 0:
            payload, ea1 = _tc_edge_first(gsrc, eap, nW1a, nb1r, nW1b,
                                          nW2, nb2d, tile_m, sel_m, Wc[l],
                                          4096)
        else:
            payload = _tc_edge(gsrc, ea1, nW1b, nW2, nb2d,
                               tile_m, sel_m, Wc[l], 4096)
        agg2 = _sc_scatter(payload, dstp, zinit)
        table, posd = _tc_update(table, agg2, Wr[l], br[l].reshape(1, -1), tn)

    out = _tc_set2set(table, batch.reshape(-1, 1).astype(jnp.int32),
                      W_ih, W_hh, b_lstm.reshape(1, -1), Wo1,
                      bo1.reshape(1, -1), Wo2, bo2.reshape(1, 1),
                      bgraph, msteps)
    return out.reshape(-1)
